# baseline (speedup 1.0000x reference)
import functools

import jax
import jax.numpy as jnp
from jax import lax
from jax.experimental import pallas as pl
from jax.experimental.pallas import tpu as pltpu
from jax.experimental.pallas import tpu_sc as plsc

N_BUCKETS = 1_000_000
DIM = 64
NQ = 128
K = 8
EPS = 1e-8
BLK = 16384
NBLK = (N_BUCKETS + BLK - 1) // BLK


def _topk_body(q_ref, e_ref, ts_ref, ti_ref, w_ref, s_ref, rv_ref, ri_ref):
    i = pl.program_id(0)

    @pl.when(i == 0)
    def _init():
        rv_ref[...] = jnp.full((NQ, K), -jnp.inf, jnp.float32)
        ri_ref[...] = jnp.zeros((NQ, K), jnp.int32)

    q = q_ref[...]
    qn = q / (jnp.sqrt(jnp.sum(q * q, axis=1, keepdims=True)) + EPS)
    e = e_ref[...]
    en = e / (jnp.sqrt(jnp.sum(e * e, axis=1, keepdims=True)) + EPS)
    s = lax.dot_general(qn, en, (((1,), (1,)), ((), ())),
                        preferred_element_type=jnp.float32)

    col = lax.broadcasted_iota(jnp.int32, (NQ, BLK), 1)
    gidx = col + i * BLK
    s = jnp.where(gidx < N_BUCKETS, s, -jnp.inf)
    s_ref[...] = s

    thr = jnp.min(rv_ref[...], axis=1)
    cnt = jnp.sum(jnp.where(s > thr[:, None], 1, 0), axis=1)
    need = jnp.minimum(jnp.max(cnt), K)

    k8 = lax.broadcasted_iota(jnp.int32, (NQ, K), 1)
    big = jnp.int32(2147483647)
    for t in range(K):
        @pl.when(t < need)
        def _extract():
            sc = s_ref[...]
            bm = jnp.max(sc, axis=1)
            eqm = sc == bm[:, None]
            bpos = jnp.min(jnp.where(eqm, gidx, big), axis=1)
            s_ref[...] = jnp.where(eqm, -jnp.inf, sc)
            rv = rv_ref[...]
            ri = ri_ref[...]
            rmin = jnp.min(rv, axis=1)
            rpos = jnp.min(jnp.where(rv == rmin[:, None], k8, K), axis=1)
            upd = bm > rmin
            sel = (k8 == rpos[:, None]) & upd[:, None]
            rv_ref[...] = jnp.where(sel, bm[:, None], rv)
            ri_ref[...] = jnp.where(sel, bpos[:, None], ri)

    @pl.when(i == NBLK - 1)
    def _finalize():
        v = rv_ref[...]
        idx = ri_ref[...]
        ts = jnp.zeros((NQ, K), jnp.float32)
        ti = jnp.zeros((NQ, K), jnp.int32)
        for t in range(K):
            m = jnp.max(v, axis=1)
            p = jnp.min(jnp.where(v == m[:, None], k8, K), axis=1)
            oh = k8 == p[:, None]
            ts = jnp.where(k8 == t, m[:, None], ts)
            ti = jnp.where(k8 == t,
                           jnp.sum(jnp.where(oh, idx, 0), axis=1)[:, None], ti)
            v = jnp.where(oh, -jnp.inf, v)
        ex = jnp.exp(ts - ts[:, 0:1])
        w = ex / jnp.sum(ex, axis=1, keepdims=True)
        ts_ref[...] = ts
        ti_ref[...] = ti
        w_ref[...] = w


def _topk_call(query, bucket_embeddings):
    return pl.pallas_call(
        _topk_body,
        grid=(NBLK,),
        in_specs=[
            pl.BlockSpec((NQ, DIM), lambda i: (0, 0)),
            pl.BlockSpec((BLK, DIM), lambda i: (i, 0)),
        ],
        out_specs=[
            pl.BlockSpec((NQ, K), lambda i: (0, 0)),
            pl.BlockSpec((NQ, K), lambda i: (0, 0)),
            pl.BlockSpec((NQ, K), lambda i: (0, 0)),
        ],
        out_shape=[
            jax.ShapeDtypeStruct((NQ, K), jnp.float32),
            jax.ShapeDtypeStruct((NQ, K), jnp.int32),
            jax.ShapeDtypeStruct((NQ, K), jnp.float32),
        ],
        scratch_shapes=[
            pltpu.VMEM((NQ, BLK), jnp.float32),
            pltpu.VMEM((NQ, K), jnp.float32),
            pltpu.VMEM((NQ, K), jnp.int32),
        ],
        compiler_params=pltpu.CompilerParams(
            dimension_semantics=("arbitrary",),
        ),
    )(query, bucket_embeddings)


def _gather_weighted(bucket_values, idx_flat, w_flat):
    info = plsc.get_sparse_core_info()
    nc, ns = info.num_cores, info.num_subcores
    nw = nc * ns
    qpw = NQ // nw
    rpw = qpw * K
    mesh = plsc.VectorSubcoreMesh(core_axis_name="c", subcore_axis_name="s")
    values2 = bucket_values.reshape(N_BUCKETS // 2, 2 * DIM)

    @functools.partial(
        pl.kernel,
        mesh=mesh,
        out_type=jax.ShapeDtypeStruct((NQ, DIM), jnp.float32),
        scratch_types=[
            pltpu.VMEM((rpw,), jnp.int32),
            pltpu.VMEM((rpw,), jnp.int32),
            pltpu.VMEM((rpw,), jnp.float32),
            pltpu.VMEM((rpw, 2 * DIM), jnp.float32),
            pltpu.VMEM((qpw, DIM), jnp.float32),
            pltpu.SemaphoreType.DMA,
        ],
    )
    def gather_kernel(values_hbm, idx_hbm, w_hbm, out_hbm,
                      idx_v, g_v, w_v, rows_v, out_v, sem):
        wid = lax.axis_index("s") * nc + lax.axis_index("c")
        base = wid * rpw
        pltpu.sync_copy(idx_hbm.at[pl.ds(base, rpw)], idx_v)
        pltpu.sync_copy(w_hbm.at[pl.ds(base, rpw)], w_v)
        for j in range(rpw // 16):
            g_v[pl.ds(j * 16, 16)] = lax.shift_right_logical(
                idx_v[pl.ds(j * 16, 16)], 1)
        pltpu.async_copy(values_hbm.at[g_v], rows_v, sem).wait()
        dnums = lax.GatherDimensionNumbers(
            offset_dims=(), collapsed_slice_dims=(0,), start_index_map=(0,))

        def bcast(vec16, j):
            return lax.gather(
                vec16, jnp.full((16, 1), j, jnp.int32), dnums,
                slice_sizes=(1,),
                mode=lax.GatherScatterMode.PROMISE_IN_BOUNDS)

        for q in range(qpw):
            half = (q * K // 16) * 16
            wrow = w_v[pl.ds(half, 16)]
            irow = idx_v[pl.ds(half, 16)]
            wvs = [bcast(wrow, (q * K) % 16 + k) for k in range(K)]
            pfs = [(bcast(irow, (q * K) % 16 + k) & 1).astype(jnp.float32)
                   for k in range(K)]
            for c in range(DIM // 16):
                acc = jnp.zeros((16,), jnp.float32)
                for k in range(K):
                    r = q * K + k
                    lo = rows_v[r, pl.ds(c * 16, 16)]
                    hi = rows_v[r, pl.ds(DIM + c * 16, 16)]
                    acc = acc + wvs[k] * (lo + pfs[k] * (hi - lo))
                out_v[q, pl.ds(c * 16, 16)] = acc
        pltpu.sync_copy(out_v, out_hbm.at[pl.ds(wid * qpw, qpw)])

    return gather_kernel(values2, idx_flat, w_flat)


def kernel(query, bucket_embeddings, bucket_values):
    ts, ti, w = _topk_call(query, bucket_embeddings)
    retrieved = _gather_weighted(bucket_values, ti.reshape(-1), w.reshape(-1))
    return retrieved, ts, ti


# speedup vs baseline: 1.0576x; 1.0576x over previous
import functools

import jax
import jax.numpy as jnp
from jax import lax
from jax.experimental import pallas as pl
from jax.experimental.pallas import tpu as pltpu
from jax.experimental.pallas import tpu_sc as plsc

N_BUCKETS = 1_000_000
DIM = 64
NQ = 128
K = 8
EPS = 1e-8
BLK = 8192
NBLK = (N_BUCKETS + BLK - 1) // BLK


def _topk_body(q_ref, e_ref, ts_ref, ti_ref, w_ref, s_ref, rv_ref, ri_ref):
    i = pl.program_id(0)

    @pl.when(i == 0)
    def _init():
        rv_ref[...] = jnp.full((NQ, K), -jnp.inf, jnp.float32)
        ri_ref[...] = jnp.zeros((NQ, K), jnp.int32)

    q = q_ref[...]
    qn = q / (jnp.sqrt(jnp.sum(q * q, axis=1, keepdims=True)) + EPS)
    e = e_ref[...]
    en = e / (jnp.sqrt(jnp.sum(e * e, axis=1, keepdims=True)) + EPS)
    s = lax.dot_general(qn, en, (((1,), (1,)), ((), ())),
                        preferred_element_type=jnp.float32)

    col = lax.broadcasted_iota(jnp.int32, (NQ, BLK), 1)
    gidx = col + i * BLK
    s = jnp.where(gidx < N_BUCKETS, s, -jnp.inf)
    s_ref[...] = s

    thr = jnp.min(rv_ref[...], axis=1)
    cnt = jnp.sum(jnp.where(s > thr[:, None], 1, 0), axis=1)
    need = jnp.minimum(jnp.max(cnt), K)

    k8 = lax.broadcasted_iota(jnp.int32, (NQ, K), 1)
    big = jnp.int32(2147483647)
    for t in range(K):
        @pl.when(t < need)
        def _extract():
            sc = s_ref[...]
            bm = jnp.max(sc, axis=1)
            eqm = sc == bm[:, None]
            bpos = jnp.min(jnp.where(eqm, gidx, big), axis=1)
            s_ref[...] = jnp.where(eqm, -jnp.inf, sc)
            rv = rv_ref[...]
            ri = ri_ref[...]
            rmin = jnp.min(rv, axis=1)
            rpos = jnp.min(jnp.where(rv == rmin[:, None], k8, K), axis=1)
            upd = bm > rmin
            sel = (k8 == rpos[:, None]) & upd[:, None]
            rv_ref[...] = jnp.where(sel, bm[:, None], rv)
            ri_ref[...] = jnp.where(sel, bpos[:, None], ri)

    @pl.when(i == NBLK - 1)
    def _finalize():
        v = rv_ref[...]
        idx = ri_ref[...]
        ts = jnp.zeros((NQ, K), jnp.float32)
        ti = jnp.zeros((NQ, K), jnp.int32)
        for t in range(K):
            m = jnp.max(v, axis=1)
            p = jnp.min(jnp.where(v == m[:, None], k8, K), axis=1)
            oh = k8 == p[:, None]
            ts = jnp.where(k8 == t, m[:, None], ts)
            ti = jnp.where(k8 == t,
                           jnp.sum(jnp.where(oh, idx, 0), axis=1)[:, None], ti)
            v = jnp.where(oh, -jnp.inf, v)
        ex = jnp.exp(ts - ts[:, 0:1])
        w = ex / jnp.sum(ex, axis=1, keepdims=True)
        ts_ref[...] = ts
        ti_ref[...] = ti
        w_ref[...] = w


def _topk_call(query, bucket_embeddings):
    return pl.pallas_call(
        _topk_body,
        grid=(NBLK,),
        in_specs=[
            pl.BlockSpec((NQ, DIM), lambda i: (0, 0)),
            pl.BlockSpec((BLK, DIM), lambda i: (i, 0)),
        ],
        out_specs=[
            pl.BlockSpec((NQ, K), lambda i: (0, 0)),
            pl.BlockSpec((NQ, K), lambda i: (0, 0)),
            pl.BlockSpec((NQ, K), lambda i: (0, 0)),
        ],
        out_shape=[
            jax.ShapeDtypeStruct((NQ, K), jnp.float32),
            jax.ShapeDtypeStruct((NQ, K), jnp.int32),
            jax.ShapeDtypeStruct((NQ, K), jnp.float32),
        ],
        scratch_shapes=[
            pltpu.VMEM((NQ, BLK), jnp.float32),
            pltpu.VMEM((NQ, K), jnp.float32),
            pltpu.VMEM((NQ, K), jnp.int32),
        ],
        compiler_params=pltpu.CompilerParams(
            dimension_semantics=("arbitrary",),
        ),
    )(query, bucket_embeddings)


def _gather_weighted(bucket_values, idx_flat, w_flat):
    info = plsc.get_sparse_core_info()
    nc, ns = info.num_cores, info.num_subcores
    nw = nc * ns
    qpw = NQ // nw
    rpw = qpw * K
    mesh = plsc.VectorSubcoreMesh(core_axis_name="c", subcore_axis_name="s")
    values2 = bucket_values.reshape(N_BUCKETS // 2, 2 * DIM)

    @functools.partial(
        pl.kernel,
        mesh=mesh,
        out_type=jax.ShapeDtypeStruct((NQ, DIM), jnp.float32),
        scratch_types=[
            pltpu.VMEM((rpw,), jnp.int32),
            pltpu.VMEM((rpw,), jnp.int32),
            pltpu.VMEM((rpw,), jnp.float32),
            pltpu.VMEM((rpw, 2 * DIM), jnp.float32),
            pltpu.VMEM((qpw, DIM), jnp.float32),
            pltpu.SemaphoreType.DMA,
        ],
    )
    def gather_kernel(values_hbm, idx_hbm, w_hbm, out_hbm,
                      idx_v, g_v, w_v, rows_v, out_v, sem):
        wid = lax.axis_index("s") * nc + lax.axis_index("c")
        base = wid * rpw
        pltpu.sync_copy(idx_hbm.at[pl.ds(base, rpw)], idx_v)
        pltpu.sync_copy(w_hbm.at[pl.ds(base, rpw)], w_v)
        for j in range(rpw // 16):
            g_v[pl.ds(j * 16, 16)] = lax.shift_right_logical(
                idx_v[pl.ds(j * 16, 16)], 1)
        pltpu.async_copy(values_hbm.at[g_v], rows_v, sem).wait()
        dnums = lax.GatherDimensionNumbers(
            offset_dims=(), collapsed_slice_dims=(0,), start_index_map=(0,))

        def bcast(vec16, j):
            return lax.gather(
                vec16, jnp.full((16, 1), j, jnp.int32), dnums,
                slice_sizes=(1,),
                mode=lax.GatherScatterMode.PROMISE_IN_BOUNDS)

        for q in range(qpw):
            half = (q * K // 16) * 16
            wrow = w_v[pl.ds(half, 16)]
            irow = idx_v[pl.ds(half, 16)]
            wvs = [bcast(wrow, (q * K) % 16 + k) for k in range(K)]
            pfs = [(bcast(irow, (q * K) % 16 + k) & 1).astype(jnp.float32)
                   for k in range(K)]
            for c in range(DIM // 16):
                acc = jnp.zeros((16,), jnp.float32)
                for k in range(K):
                    r = q * K + k
                    lo = rows_v[r, pl.ds(c * 16, 16)]
                    hi = rows_v[r, pl.ds(DIM + c * 16, 16)]
                    acc = acc + wvs[k] * (lo + pfs[k] * (hi - lo))
                out_v[q, pl.ds(c * 16, 16)] = acc
        pltpu.sync_copy(out_v, out_hbm.at[pl.ds(wid * qpw, qpw)])

    return gather_kernel(values2, idx_flat, w_flat)


def kernel(query, bucket_embeddings, bucket_values):
    ts, ti, w = _topk_call(query, bucket_embeddings)
    retrieved = _gather_weighted(bucket_values, ti.reshape(-1), w.reshape(-1))
    return retrieved, ts, ti
